# baseline (speedup 1.0000x reference)
import functools

import jax
import jax.numpy as jnp
from jax import lax
from jax.experimental import pallas as pl
from jax.experimental.pallas import tpu as pltpu
from jax.experimental.pallas import tpu_sc as plsc

_A = 9
_HW = 50
_P = _HW * _HW
_N = _P * _A
_NPAD = 22528
_C = 512
_PRE_N = 6000
_CN = 6016
_POST_N = 300
_NMS_T = 0.7
_MIN_SIZE = 16.0
_BIG = 1 << 30
_DEAD = -2147483648
_DUMP = 6008


def _trunk_body(xp_ref, wt_ref, hw_ref, hb_ref, cb_ref, anc_ref, lim_ref,
                locs_ref, scores_ref, y1_ref, x1_ref, y2_ref, x2_ref,
                ki_ref, rank_ref):
    f32 = jnp.float32
    jcol = jax.lax.broadcasted_iota(jnp.int32, (1, _P), 1) % _HW

    cols = []
    for t in range(9):
        dy, dx = t // 3, t % 3
        off = (dy - 1) * _HW + (dx - 1)
        xs = xp_ref[:, 51 + off: 51 + off + _P]
        if dx == 0:
            xs = jnp.where(jcol != 0, xs, 0.0)
        elif dx == 2:
            xs = jnp.where(jcol != _HW - 1, xs, 0.0)
        cols.append(xs)
    xcat = jnp.concatenate(cols, axis=0)
    acc = jnp.dot(wt_ref[...], xcat, preferred_element_type=f32)
    feat = jnp.maximum(acc + cb_ref[...], 0.0)

    heads = jnp.dot(hw_ref[...], feat, preferred_element_type=f32) + hb_ref[...]
    scores_ref[...] = heads[0:18]
    locs_ref[...] = heads[18:54]

    fg = jnp.concatenate(
        [jax.nn.sigmoid(heads[2 * a + 1:2 * a + 2] - heads[2 * a:2 * a + 1])
         for a in range(_A)], axis=0)
    loc = [jnp.concatenate([heads[18 + 4 * a + d:18 + 4 * a + d + 1]
                            for a in range(_A)], axis=0) for d in range(4)]

    ay1, ax1, ay2, ax2 = anc_ref[0], anc_ref[1], anc_ref[2], anc_ref[3]
    ah = ay2 - ay1
    aw = ax2 - ax1
    acy = ay1 + 0.5 * ah
    acx = ax1 + 0.5 * aw
    ncy = loc[0] * ah + acy
    ncx = loc[1] * aw + acx
    nh = jnp.exp(loc[2]) * ah
    nw = jnp.exp(loc[3]) * aw
    lim = lim_ref[0, 0]
    y1 = jnp.clip(ncy - 0.5 * nh, 0.0, lim)
    x1 = jnp.clip(ncx - 0.5 * nw, 0.0, lim)
    y2 = jnp.clip(ncy + 0.5 * nh, 0.0, lim)
    x2 = jnp.clip(ncx + 0.5 * nw, 0.0, lim)
    y1_ref[...] = y1
    x1_ref[...] = x1
    y2_ref[...] = y2
    x2_ref[...] = x2
    hs = y2 - y1
    ws = x2 - x1
    valid = (hs >= _MIN_SIZE) & (ws >= _MIN_SIZE)
    sc = jnp.where(valid, fg, -jnp.inf)

    kidx = (jax.lax.broadcasted_iota(jnp.int32, (_A, _P), 1) * _A
            + jax.lax.broadcasted_iota(jnp.int32, (_A, _P), 0))

    key = jax.lax.bitcast_convert_type(sc, jnp.int32)

    def bs_val(_, lohi):
        lo, hi = lohi
        mid = (lo + hi) // 2
        c = jnp.sum((key >= mid).astype(jnp.int32))
        take = c >= _PRE_N
        return jnp.where(take, mid, lo), jnp.where(take, hi, mid)

    vlo, _ = jax.lax.fori_loop(
        0, 31, bs_val, (jnp.int32(-8388609), jnp.int32(1065353218)))
    vk = vlo
    count_gt = jnp.sum((key > vk).astype(jnp.int32))
    need = _PRE_N - count_gt
    eq = key == vk

    def bs_idx(_, lohi):
        lo, hi = lohi
        mid = (lo + hi) // 2
        c = jnp.sum((eq & (kidx < mid)).astype(jnp.int32))
        take = c >= need
        return jnp.where(take, lo, mid), jnp.where(take, mid, hi)

    _, khi = jax.lax.fori_loop(
        0, 16, bs_idx, (jnp.int32(0), jnp.int32(_N + 1)))
    member = (key > vk) | (eq & (kidx < khi))

    mi = member.astype(jnp.int32)
    colcnt = jnp.sum(mi, axis=0, keepdims=True)
    inc = colcnt
    sft = 1
    while sft < _P:
        shifted = jnp.concatenate(
            [jnp.zeros((1, sft), jnp.int32), inc[:, :_P - sft]], axis=1)
        inc = inc + shifted
        sft *= 2
    pcol = inc - colcnt
    rows = [jnp.zeros((1, _P), jnp.int32)]
    run = jnp.zeros((1, _P), jnp.int32)
    for a in range(1, _A):
        run = run + mi[a - 1:a]
        rows.append(run)
    wa = jnp.concatenate(rows, axis=0)
    rank = pcol + wa
    rank_ref[...] = jnp.where(member, rank, jnp.int32(_DUMP))
    ki_ref[...] = jnp.where(member, key, jnp.int32(_DEAD))


def _nms_body(c4_ref, kc_ref, rois_ref):
    f32 = jnp.float32
    y1 = c4_ref[0]
    x1 = c4_ref[1]
    y2 = c4_ref[2]
    x2 = c4_ref[3]
    pos = (jax.lax.broadcasted_iota(jnp.int32, (47, 128), 0) * 128
           + jax.lax.broadcasted_iota(jnp.int32, (47, 128), 1))
    ki0 = jnp.where(pos < _PRE_N, kc_ref[...], jnp.int32(_DEAD))
    area = (y2 - y1) * (x2 - x1)
    lane = jax.lax.broadcasted_iota(jnp.int32, (1, 128), 1)

    def nms_body(i, st):
        ki, pad = st
        m = jnp.max(ki)
        has = m > _DEAD
        eqm = ki == m
        cnt = jnp.sum(eqm.astype(jnp.int32))

        def msum(arr, msk):
            return jnp.sum(jnp.where(msk, arr, 0.0))

        ry1, rx1, ry2, rx2 = (msum(y1, eqm), msum(x1, eqm),
                              msum(y2, eqm), msum(x2, eqm))

        def fast():
            return ry1, rx1, ry2, rx2, jnp.where(eqm, jnp.int32(_DEAD), ki)

        def slow():
            selpos = jnp.min(jnp.where(eqm, pos, jnp.int32(_BIG)))
            selm = pos == selpos
            return (msum(y1, selm), msum(x1, selm), msum(y2, selm),
                    msum(x2, selm), jnp.where(selm, jnp.int32(_DEAD), ki))

        by1, bx1, by2, bx2, kiex = jax.lax.cond(cnt == 1, fast, slow)
        barea = (by2 - by1) * (bx2 - bx1)
        iy1 = jnp.maximum(by1, y1)
        ix1 = jnp.maximum(bx1, x1)
        iy2 = jnp.minimum(by2, y2)
        ix2 = jnp.minimum(bx2, x2)
        inter = jnp.maximum(iy2 - iy1, 0.0) * jnp.maximum(ix2 - ix1, 0.0)
        iou = inter / (barea + area - inter + 1e-9)
        keep = (iou <= _NMS_T) & has
        ki = jnp.where(keep, kiex, jnp.int32(_DEAD))
        row = jnp.where(lane == 0, by1,
                        jnp.where(lane == 1, bx1,
                                  jnp.where(lane == 2, by2,
                                            jnp.where(lane == 3, bx2, 0.0))))
        pad = jnp.where(i == 0, row, pad)
        rois_ref[pl.ds(i, 1), :] = jnp.where(has, row, pad)
        return ki, pad

    jax.lax.fori_loop(0, _POST_N, nms_body,
                      (ki0, jnp.zeros((1, 128), f32)))


def _make_compact():
    info = plsc.get_sparse_core_info()
    ns = info.num_subcores
    mesh = plsc.VectorSubcoreMesh(core_axis_name="c", subcore_axis_name="s")

    @functools.partial(
        pl.kernel, mesh=mesh,
        compiler_params=pltpu.CompilerParams(needs_layout_passes=False),
        out_type=[jax.ShapeDtypeStruct((4, _CN), jnp.float32),
                  jax.ShapeDtypeStruct((_CN,), jnp.int32)],
        scratch_types=[pltpu.VMEM((_NPAD,), jnp.float32),
                       pltpu.VMEM((_NPAD,), jnp.int32),
                       pltpu.VMEM((_NPAD,), jnp.int32),
                       pltpu.VMEM((_CN,), jnp.float32),
                       pltpu.VMEM((_CN,), jnp.int32)])
    def compact(pay_hbm, key_hbm, rank_hbm, out4_hbm, outk_hbm,
                vf, vkey, vrank, of, ok):
        wid = lax.axis_index("c") * ns + lax.axis_index("s")

        for j in range(4):
            @pl.when(wid == j)
            def _(j=j):
                pltpu.sync_copy(rank_hbm, vrank)
                pltpu.sync_copy(pay_hbm.at[j], vf)

                def body(i, carry):
                    sl = pl.ds(i * 16, 16)
                    plsc.store_scatter(of, [vrank[sl]], vf[sl])
                    return carry

                lax.fori_loop(0, _NPAD // 16, body, 0)
                pltpu.sync_copy(of, out4_hbm.at[j])

        @pl.when(wid == 4)
        def _():
            pltpu.sync_copy(rank_hbm, vrank)
            pltpu.sync_copy(key_hbm, vkey)

            def body(i, carry):
                sl = pl.ds(i * 16, 16)
                plsc.store_scatter(ok, [vrank[sl]], vkey[sl])
                return carry

            lax.fori_loop(0, _NPAD // 16, body, 0)
            pltpu.sync_copy(ok, outk_hbm)

    return compact


def kernel(x, img_size, conv1_w, conv1_b, score_w, score_b, loc_w, loc_b,
           anchors):
    x2 = x.reshape(_C, _P)
    xp = jnp.pad(x2, ((0, 0), (51, 51)))
    wt = conv1_w.transpose(0, 2, 3, 1).reshape(_C, 9 * _C)
    hw = jnp.concatenate([score_w.reshape(2 * _A, _C),
                          loc_w.reshape(4 * _A, _C)], axis=0)
    hb = jnp.concatenate([score_b, loc_b]).reshape(6 * _A, 1)
    cb = conv1_b.reshape(_C, 1)
    anc = anchors.reshape(_P, _A, 4).transpose(2, 1, 0)
    lim = jnp.asarray(img_size, jnp.float32).reshape(1, 1)

    f32 = jnp.float32
    locs_o, scores_o, y1o, x1o, y2o, x2o, kio, ranko = pl.pallas_call(
        _trunk_body,
        out_shape=[
            jax.ShapeDtypeStruct((4 * _A, _P), f32),
            jax.ShapeDtypeStruct((2 * _A, _P), f32),
            jax.ShapeDtypeStruct((_A, _P), f32),
            jax.ShapeDtypeStruct((_A, _P), f32),
            jax.ShapeDtypeStruct((_A, _P), f32),
            jax.ShapeDtypeStruct((_A, _P), f32),
            jax.ShapeDtypeStruct((_A, _P), jnp.int32),
            jax.ShapeDtypeStruct((_A, _P), jnp.int32),
        ],
    )(xp, wt, hw, hb, cb, anc, lim)

    pay = jnp.stack([y1o.reshape(_N), x1o.reshape(_N),
                     y2o.reshape(_N), x2o.reshape(_N)], axis=0)
    pay = jnp.pad(pay, ((0, 0), (0, _NPAD - _N)))
    ki = jnp.pad(kio.reshape(_N), (0, _NPAD - _N),
                 constant_values=_DEAD)
    rank = jnp.pad(ranko.reshape(_N), (0, _NPAD - _N),
                   constant_values=_DUMP)

    out4, outk = _make_compact()(pay, ki, rank)

    rois_o = pl.pallas_call(
        _nms_body,
        out_shape=[jax.ShapeDtypeStruct((_POST_N, 128), f32)],
    )(out4.reshape(4, 47, 128), outk.reshape(47, 128))[0]

    rpn_locs = locs_o.T.reshape(1, _N, 4)
    rpn_scores = scores_o.T.reshape(1, _N, 2)
    rois = rois_o[:, :4].reshape(1, _POST_N, 4)
    return rpn_locs, rpn_scores, rois


# speedup vs baseline: 1.0681x; 1.0681x over previous
import functools

import jax
import jax.numpy as jnp
from jax import lax
from jax.experimental import pallas as pl
from jax.experimental.pallas import tpu as pltpu
from jax.experimental.pallas import tpu_sc as plsc

_A = 9
_HW = 50
_P = _HW * _HW
_N = _P * _A
_NPAD = 22528
_C = 512
_PRE_N = 6000
_CN = 6016
_POST_N = 300
_NMS_T = 0.7
_MIN_SIZE = 16.0
_BIG = 1 << 30
_DEAD = -2147483648
_DUMP = 6008


def _trunk_body(xp_ref, wt_ref, hw_ref, hb_ref, cb_ref, anc_ref, lim_ref,
                locs_ref, scores_ref, buf5_ref, rank_ref):
    f32 = jnp.float32
    jcol = jax.lax.broadcasted_iota(jnp.int32, (1, _P), 1) % _HW

    cols = []
    for t in range(9):
        dy, dx = t // 3, t % 3
        off = (dy - 1) * _HW + (dx - 1)
        if off < 0:
            xs = jnp.concatenate(
                [jnp.zeros((_C, -off), f32), xp_ref[:, :_P + off]], axis=1)
        elif off > 0:
            xs = jnp.concatenate(
                [xp_ref[:, off:], jnp.zeros((_C, off), f32)], axis=1)
        else:
            xs = xp_ref[...]
        if dx == 0:
            xs = jnp.where(jcol != 0, xs, 0.0)
        elif dx == 2:
            xs = jnp.where(jcol != _HW - 1, xs, 0.0)
        cols.append(xs)
    xcat = jnp.concatenate(cols, axis=0)
    acc = jnp.dot(wt_ref[...], xcat, preferred_element_type=f32)
    feat = jnp.maximum(acc + cb_ref[...], 0.0)

    heads = jnp.dot(hw_ref[...], feat, preferred_element_type=f32) + hb_ref[...]
    scores_ref[...] = heads[0:18]
    locs_ref[...] = heads[18:54]

    fg = jnp.concatenate(
        [jax.nn.sigmoid(heads[2 * a + 1:2 * a + 2] - heads[2 * a:2 * a + 1])
         for a in range(_A)], axis=0)
    loc = [jnp.concatenate([heads[18 + 4 * a + d:18 + 4 * a + d + 1]
                            for a in range(_A)], axis=0) for d in range(4)]

    ay1, ax1, ay2, ax2 = anc_ref[0], anc_ref[1], anc_ref[2], anc_ref[3]
    ah = ay2 - ay1
    aw = ax2 - ax1
    acy = ay1 + 0.5 * ah
    acx = ax1 + 0.5 * aw
    ncy = loc[0] * ah + acy
    ncx = loc[1] * aw + acx
    nh = jnp.exp(loc[2]) * ah
    nw = jnp.exp(loc[3]) * aw
    lim = lim_ref[0, 0]
    y1 = jnp.clip(ncy - 0.5 * nh, 0.0, lim)
    x1 = jnp.clip(ncx - 0.5 * nw, 0.0, lim)
    y2 = jnp.clip(ncy + 0.5 * nh, 0.0, lim)
    x2 = jnp.clip(ncx + 0.5 * nw, 0.0, lim)
    hs = y2 - y1
    ws = x2 - x1
    valid = (hs >= _MIN_SIZE) & (ws >= _MIN_SIZE)
    sc = jnp.where(valid, fg, -jnp.inf)

    kidx = (jax.lax.broadcasted_iota(jnp.int32, (_A, _P), 1) * _A
            + jax.lax.broadcasted_iota(jnp.int32, (_A, _P), 0))

    key = jax.lax.bitcast_convert_type(sc, jnp.int32)

    def bs_val(_, lohi):
        lo, hi = lohi
        mid = (lo + hi) // 2
        c = jnp.sum((key >= mid).astype(jnp.int32))
        take = c >= _PRE_N
        return jnp.where(take, mid, lo), jnp.where(take, hi, mid)

    vlo, _ = jax.lax.fori_loop(
        0, 31, bs_val, (jnp.int32(-8388609), jnp.int32(1065353218)))
    vk = vlo
    count_gt = jnp.sum((key > vk).astype(jnp.int32))
    need = _PRE_N - count_gt
    eq = key == vk

    def bs_idx(_, lohi):
        lo, hi = lohi
        mid = (lo + hi) // 2
        c = jnp.sum((eq & (kidx < mid)).astype(jnp.int32))
        take = c >= need
        return jnp.where(take, lo, mid), jnp.where(take, mid, hi)

    _, khi = jax.lax.fori_loop(
        0, 16, bs_idx, (jnp.int32(0), jnp.int32(_N + 1)))
    member = (key > vk) | (eq & (kidx < khi))

    mi = member.astype(jnp.int32)
    colcnt = jnp.sum(mi, axis=0, keepdims=True)
    inc = colcnt
    sft = 1
    while sft < _P:
        shifted = jnp.concatenate(
            [jnp.zeros((1, sft), jnp.int32), inc[:, :_P - sft]], axis=1)
        inc = inc + shifted
        sft *= 2
    pcol = inc - colcnt
    rows = [jnp.zeros((1, _P), jnp.int32)]
    run = jnp.zeros((1, _P), jnp.int32)
    for a in range(1, _A):
        run = run + mi[a - 1:a]
        rows.append(run)
    wa = jnp.concatenate(rows, axis=0)
    rank = pcol + wa
    rank_ref[...] = jnp.where(member, rank, jnp.int32(_DUMP))
    ki = jnp.where(member, key, jnp.int32(_DEAD))
    kif = jax.lax.bitcast_convert_type(ki, f32)
    buf5_ref[...] = jnp.concatenate(
        [v.reshape(1, _A, _P) for v in (y1, x1, y2, x2, kif)], axis=0)


def _nms_body(c5_ref, rois_ref):
    f32 = jnp.float32
    y1 = c5_ref[0]
    x1 = c5_ref[1]
    y2 = c5_ref[2]
    x2 = c5_ref[3]
    kc = jax.lax.bitcast_convert_type(c5_ref[4], jnp.int32)
    pos = (jax.lax.broadcasted_iota(jnp.int32, (47, 128), 0) * 128
           + jax.lax.broadcasted_iota(jnp.int32, (47, 128), 1))
    ki0 = jnp.where(pos < _PRE_N, kc, jnp.int32(_DEAD))
    area = (y2 - y1) * (x2 - x1)
    lane = jax.lax.broadcasted_iota(jnp.int32, (1, 128), 1)

    def nms_body(i, st):
        ki, pad = st
        m = jnp.max(ki)
        has = m > _DEAD
        eqm = ki == m
        cnt = jnp.sum(eqm.astype(jnp.int32))

        def msum(arr, msk):
            return jnp.sum(jnp.where(msk, arr, 0.0))

        ry1, rx1, ry2, rx2 = (msum(y1, eqm), msum(x1, eqm),
                              msum(y2, eqm), msum(x2, eqm))

        def fast():
            return ry1, rx1, ry2, rx2, jnp.where(eqm, jnp.int32(_DEAD), ki)

        def slow():
            selpos = jnp.min(jnp.where(eqm, pos, jnp.int32(_BIG)))
            selm = pos == selpos
            return (msum(y1, selm), msum(x1, selm), msum(y2, selm),
                    msum(x2, selm), jnp.where(selm, jnp.int32(_DEAD), ki))

        by1, bx1, by2, bx2, kiex = jax.lax.cond(cnt == 1, fast, slow)
        barea = (by2 - by1) * (bx2 - bx1)
        iy1 = jnp.maximum(by1, y1)
        ix1 = jnp.maximum(bx1, x1)
        iy2 = jnp.minimum(by2, y2)
        ix2 = jnp.minimum(bx2, x2)
        inter = jnp.maximum(iy2 - iy1, 0.0) * jnp.maximum(ix2 - ix1, 0.0)
        iou = inter / (barea + area - inter + 1e-9)
        keep = (iou <= _NMS_T) & has
        ki = jnp.where(keep, kiex, jnp.int32(_DEAD))
        row = jnp.where(lane == 0, by1,
                        jnp.where(lane == 1, bx1,
                                  jnp.where(lane == 2, by2,
                                            jnp.where(lane == 3, bx2, 0.0))))
        pad = jnp.where(i == 0, row, pad)
        rois_ref[pl.ds(i, 1), :] = jnp.where(has, row, pad)
        return ki, pad

    jax.lax.fori_loop(0, _POST_N, nms_body,
                      (ki0, jnp.zeros((1, 128), f32)))


def _make_compact():
    info = plsc.get_sparse_core_info()
    ns = info.num_subcores
    mesh = plsc.VectorSubcoreMesh(core_axis_name="c", subcore_axis_name="s")

    @functools.partial(
        pl.kernel, mesh=mesh,
        compiler_params=pltpu.CompilerParams(needs_layout_passes=False),
        out_type=jax.ShapeDtypeStruct((5 * _CN,), jnp.float32),
        scratch_types=[pltpu.VMEM((_NPAD,), jnp.float32),
                       pltpu.VMEM((_NPAD,), jnp.int32),
                       pltpu.VMEM((_CN,), jnp.float32)])
    def compact(pay_hbm, rank_hbm, out5_hbm, vf, vrank, of):
        wid = lax.axis_index("c") * ns + lax.axis_index("s")

        for j in range(5):
            @pl.when(wid == j)
            def _(j=j):
                pltpu.sync_copy(rank_hbm, vrank)
                pltpu.sync_copy(pay_hbm.at[pl.ds(j * _NPAD, _NPAD)], vf)

                def body(i, carry):
                    sl = pl.ds(i * 16, 16)
                    plsc.store_scatter(of, [vrank[sl]], vf[sl])
                    return carry

                lax.fori_loop(0, _NPAD // 16, body, 0)
                pltpu.sync_copy(of, out5_hbm.at[pl.ds(j * _CN, _CN)])

    return compact


def kernel(x, img_size, conv1_w, conv1_b, score_w, score_b, loc_w, loc_b,
           anchors):
    xp = x.reshape(_C, _P)
    wt = conv1_w.transpose(0, 2, 3, 1).reshape(_C, 9 * _C)
    hw = jnp.concatenate([score_w.reshape(2 * _A, _C),
                          loc_w.reshape(4 * _A, _C)], axis=0)
    hb = jnp.concatenate([score_b, loc_b]).reshape(6 * _A, 1)
    cb = conv1_b.reshape(_C, 1)
    anc = anchors.reshape(_P, _A, 4).transpose(2, 1, 0)
    lim = jnp.asarray(img_size, jnp.float32).reshape(1, 1)

    f32 = jnp.float32
    locs_o, scores_o, buf5, ranko = pl.pallas_call(
        _trunk_body,
        out_shape=[
            jax.ShapeDtypeStruct((4 * _A, _P), f32),
            jax.ShapeDtypeStruct((2 * _A, _P), f32),
            jax.ShapeDtypeStruct((5, _A, _P), f32),
            jax.ShapeDtypeStruct((_A, _P), jnp.int32),
        ],
    )(xp, wt, hw, hb, cb, anc, lim)

    pay = jnp.pad(buf5.reshape(5, _N),
                  ((0, 0), (0, _NPAD - _N))).reshape(5 * _NPAD)
    rank = jnp.pad(ranko.reshape(_N), (0, _NPAD - _N),
                   constant_values=_DUMP)

    out5 = _make_compact()(pay, rank)

    rois_o = pl.pallas_call(
        _nms_body,
        out_shape=[jax.ShapeDtypeStruct((_POST_N, 128), f32)],
    )(out5.reshape(5, 47, 128))[0]

    rpn_locs = locs_o.T.reshape(1, _N, 4)
    rpn_scores = scores_o.T.reshape(1, _N, 2)
    rois = rois_o[:, :4].reshape(1, _POST_N, 4)
    return rpn_locs, rpn_scores, rois
